# baseline (speedup 1.0000x reference)
import functools

import jax
import jax.numpy as jnp
from jax import lax
from jax.experimental import pallas as pl
from jax.experimental.pallas import tpu as pltpu
from jax.experimental.pallas import tpu_sc as plsc

BATCH = 16384
EMBED_DIM = 16
NUM_NUMERIC = 6
EPS = 1e-5

@functools.cache
def _make_sc_gather():
    info = plsc.get_sparse_core_info()
    nc, ns = info.num_cores, info.num_subcores
    b_per_w = BATCH // (nc * ns)
    mesh = plsc.VectorSubcoreMesh(core_axis_name="c", subcore_axis_name="s")

    @functools.partial(
        pl.kernel,
        mesh=mesh,
        out_type=jax.ShapeDtypeStruct((BATCH, EMBED_DIM), jnp.float32),
        scratch_types=[
            pltpu.VMEM((b_per_w,), jnp.int32),
            pltpu.VMEM((b_per_w, EMBED_DIM), jnp.float32),
            pltpu.SemaphoreType.DMA,
        ],
        compiler_params=pltpu.CompilerParams(use_tc_tiling_on_sc=False),
    )
    def _sc_gather(table_hbm, idx_hbm, out_hbm, idx_v, rows_v, sem):
        wid = lax.axis_index("s") * nc + lax.axis_index("c")
        base = wid * b_per_w
        pltpu.sync_copy(idx_hbm.at[pl.ds(base, b_per_w)], idx_v)
        pltpu.async_copy(table_hbm.at[idx_v], rows_v, sem).wait()
        pltpu.sync_copy(rows_v, out_hbm.at[pl.ds(base, b_per_w)])

    return _sc_gather


def _mlp_body(g_ref, nf_ref, w1e_ref, w1n_ref, b1_ref, g1_ref, be1_ref,
              w2_ref, b2_ref, g2_ref, be2_ref, w3_ref, b3_ref, out_ref):
    h = jnp.dot(g_ref[...], w1e_ref[...], preferred_element_type=jnp.float32)
    h = h + jnp.dot(nf_ref[...], w1n_ref[...], preferred_element_type=jnp.float32)
    h = jnp.maximum(h + b1_ref[...], 0.0)
    mean = jnp.mean(h, axis=0, keepdims=True)
    var = jnp.mean((h - mean) ** 2, axis=0, keepdims=True)
    h = g1_ref[...] * ((h - mean) / jnp.sqrt(var + EPS)) + be1_ref[...]
    h = jnp.dot(h, w2_ref[...], preferred_element_type=jnp.float32)
    h = jnp.maximum(h + b2_ref[...], 0.0)
    mean2 = jnp.mean(h, axis=0, keepdims=True)
    var2 = jnp.mean((h - mean2) ** 2, axis=0, keepdims=True)
    h = g2_ref[...] * ((h - mean2) / jnp.sqrt(var2 + EPS)) + be2_ref[...]
    out_ref[...] = (
        jnp.dot(h, w3_ref[...], preferred_element_type=jnp.float32) + b3_ref[...]
    )


def kernel(channel_ids, numeric_features, emb, W1, b1, g1, be1,
           W2, b2, g2, be2, W3, b3):
    gathered = _make_sc_gather()(emb, channel_ids)
    out = pl.pallas_call(
        _mlp_body,
        out_shape=jax.ShapeDtypeStruct((BATCH, 1), jnp.float32),
    )(
        gathered,
        numeric_features,
        W1[:, :EMBED_DIM].T,
        W1[:, EMBED_DIM:].T,
        b1.reshape(1, -1),
        g1.reshape(1, -1),
        be1.reshape(1, -1),
        W2.T,
        b2.reshape(1, -1),
        g2.reshape(1, -1),
        be2.reshape(1, -1),
        W3.T,
        b3.reshape(1, 1),
    )
    return out[:, 0]


# speedup vs baseline: 4.5615x; 4.5615x over previous
import functools

import jax
import jax.numpy as jnp
from jax import lax
from jax.experimental import pallas as pl
from jax.experimental.pallas import tpu as pltpu
from jax.experimental.pallas import tpu_sc as plsc

BATCH = 16384
EMBED_DIM = 16
NUM_NUMERIC = 6
EPS = 1e-5
ROWS_PER_BLOCK = 8
BLK = ROWS_PER_BLOCK * EMBED_DIM
LANES = 16
NUM_CH = 1000000
RUN_SHIFT = 12
RUN = 1 << RUN_SHIFT
T_CH = RUN * ROWS_PER_BLOCK
N_TSTEPS = -(-NUM_CH // T_CH)
NUM_BLOCKS = N_TSTEPS * RUN


def _transpose_body(in_ref, out_ref):
    x = in_ref[...]
    stacked = jnp.concatenate(
        [x[:, o * RUN:(o + 1) * RUN] for o in range(ROWS_PER_BLOCK)], axis=0
    )
    out_ref[...] = stacked.T


@functools.cache
def _make_sc_gather():
    info = plsc.get_sparse_core_info()
    nc, ns = info.num_cores, info.num_subcores
    b_per_w = BATCH // (nc * ns)
    n_groups = b_per_w // LANES
    mesh = plsc.VectorSubcoreMesh(core_axis_name="c", subcore_axis_name="s")

    @functools.partial(
        pl.kernel,
        mesh=mesh,
        out_type=jax.ShapeDtypeStruct((BATCH, EMBED_DIM), jnp.float32),
        scratch_types=[
            pltpu.VMEM((b_per_w,), jnp.int32),
            pltpu.VMEM((b_per_w,), jnp.int32),
            pltpu.VMEM((b_per_w, EMBED_DIM), jnp.float32),
            pltpu.SemaphoreType.DMA,
        ],
        compiler_params=pltpu.CompilerParams(use_tc_tiling_on_sc=False),
    )
    def _sc_gather(table_hbm, idx_hbm, out_hbm, idx_v, row_v, rows_v, sem):
        wid = lax.axis_index("s") * nc + lax.axis_index("c")
        base = wid * b_per_w
        pltpu.sync_copy(idx_hbm.at[pl.ds(base, b_per_w)], idx_v)
        for i in range(n_groups):
            sl = pl.ds(i * LANES, LANES)
            v = idx_v[sl]
            q = lax.shift_right_logical(v, RUN_SHIFT + 3)
            o = lax.shift_right_logical(v, RUN_SHIFT) & (ROWS_PER_BLOCK - 1)
            row_v[sl] = (
                lax.shift_left(q, RUN_SHIFT + 3)
                | lax.shift_left(v & (RUN - 1), 3)
                | o
            )
        pltpu.async_copy(table_hbm.at[row_v], rows_v, sem).wait()
        pltpu.sync_copy(rows_v, out_hbm.at[pl.ds(base, b_per_w)])

    return _sc_gather


_EXTRACT_CHUNK = 4096


def _extract_body(blk_ref, ids_ref, out_ref):
    off = lax.shift_right_logical(ids_ref[...], RUN_SHIFT) & (
        ROWS_PER_BLOCK - 1)
    acc = jnp.zeros((_EXTRACT_CHUNK, EMBED_DIM), jnp.float32)
    for o in range(ROWS_PER_BLOCK):
        acc = acc + jnp.where(off == o, 1.0, 0.0) * blk_ref[
            :, o * EMBED_DIM:(o + 1) * EMBED_DIM]
    out_ref[...] = acc


def _mlp_body(g_ref, nf_ref, w1e_ref, w1n_ref, b1_ref, g1_ref, be1_ref,
              w2_ref, b2_ref, g2_ref, be2_ref, w3_ref, b3_ref, out_ref):
    h = jnp.dot(g_ref[...], w1e_ref[...], preferred_element_type=jnp.float32)
    h = h + jnp.dot(nf_ref[...], w1n_ref[...], preferred_element_type=jnp.float32)
    h = jnp.maximum(h + b1_ref[...], 0.0)
    mean = jnp.mean(h, axis=0, keepdims=True)
    var = jnp.mean((h - mean) ** 2, axis=0, keepdims=True)
    h = g1_ref[...] * ((h - mean) / jnp.sqrt(var + EPS)) + be1_ref[...]
    h = jnp.dot(h, w2_ref[...], preferred_element_type=jnp.float32)
    h = jnp.maximum(h + b2_ref[...], 0.0)
    mean2 = jnp.mean(h, axis=0, keepdims=True)
    var2 = jnp.mean((h - mean2) ** 2, axis=0, keepdims=True)
    h = g2_ref[...] * ((h - mean2) / jnp.sqrt(var2 + EPS)) + be2_ref[...]
    out_ref[...] = (
        jnp.dot(h, w3_ref[...], preferred_element_type=jnp.float32) + b3_ref[...]
    )


def kernel(channel_ids, numeric_features, emb, W1, b1, g1, be1,
           W2, b2, g2, be2, W3, b3):
    emb_t = emb.T
    table_blk = pl.pallas_call(
        _transpose_body,
        grid=(N_TSTEPS,),
        in_specs=[pl.BlockSpec((EMBED_DIM, T_CH), lambda i: (0, i))],
        out_specs=pl.BlockSpec((RUN, BLK), lambda i: (i, 0)),
        out_shape=jax.ShapeDtypeStruct((NUM_BLOCKS, BLK), jnp.float32),
    )(emb_t)
    table_rows = table_blk.reshape(NUM_BLOCKS * ROWS_PER_BLOCK, EMBED_DIM)
    gathered = _make_sc_gather()(table_rows, channel_ids)
    out = pl.pallas_call(
        _mlp_body,
        out_shape=jax.ShapeDtypeStruct((BATCH, 1), jnp.float32),
    )(
        gathered,
        numeric_features,
        W1[:, :EMBED_DIM].T,
        W1[:, EMBED_DIM:].T,
        b1.reshape(1, -1),
        g1.reshape(1, -1),
        be1.reshape(1, -1),
        W2.T,
        b2.reshape(1, -1),
        g2.reshape(1, -1),
        be2.reshape(1, -1),
        W3.T,
        b3.reshape(1, 1),
    )
    return out[:, 0]


# speedup vs baseline: 4.8539x; 1.0641x over previous
import functools

import jax
import jax.numpy as jnp
from jax import lax
from jax.experimental import pallas as pl
from jax.experimental.pallas import tpu as pltpu
from jax.experimental.pallas import tpu_sc as plsc

BATCH = 16384
EMBED_DIM = 16
NUM_NUMERIC = 6
EPS = 1e-5
ROWS_PER_BLOCK = 8
BLK = ROWS_PER_BLOCK * EMBED_DIM
LANES = 16
NUM_CH = 1000000
RUN_SHIFT = 13
RUN = 1 << RUN_SHIFT
T_CH = RUN * ROWS_PER_BLOCK
N_TSTEPS = -(-NUM_CH // T_CH)
NUM_BLOCKS = N_TSTEPS * RUN


def _transpose_body(in_ref, out_ref):
    x = in_ref[...]
    stacked = jnp.concatenate(
        [x[:, o * RUN:(o + 1) * RUN] for o in range(ROWS_PER_BLOCK)], axis=0
    )
    out_ref[...] = stacked.T


@functools.cache
def _make_sc_gather():
    info = plsc.get_sparse_core_info()
    nc, ns = info.num_cores, info.num_subcores
    b_per_w = BATCH // (nc * ns)
    n_groups = b_per_w // LANES
    mesh = plsc.VectorSubcoreMesh(core_axis_name="c", subcore_axis_name="s")

    @functools.partial(
        pl.kernel,
        mesh=mesh,
        out_type=jax.ShapeDtypeStruct((BATCH, EMBED_DIM), jnp.float32),
        scratch_types=[
            pltpu.VMEM((b_per_w,), jnp.int32),
            pltpu.VMEM((b_per_w,), jnp.int32),
            pltpu.VMEM((b_per_w, EMBED_DIM), jnp.float32),
            pltpu.SemaphoreType.DMA,
        ],
        compiler_params=pltpu.CompilerParams(use_tc_tiling_on_sc=False),
    )
    def _sc_gather(table_hbm, idx_hbm, out_hbm, idx_v, row_v, rows_v, sem):
        wid = lax.axis_index("s") * nc + lax.axis_index("c")
        base = wid * b_per_w
        pltpu.sync_copy(idx_hbm.at[pl.ds(base, b_per_w)], idx_v)
        for i in range(n_groups):
            sl = pl.ds(i * LANES, LANES)
            v = idx_v[sl]
            q = lax.shift_right_logical(v, RUN_SHIFT + 3)
            o = lax.shift_right_logical(v, RUN_SHIFT) & (ROWS_PER_BLOCK - 1)
            row_v[sl] = (
                lax.shift_left(q, RUN_SHIFT + 3)
                | lax.shift_left(v & (RUN - 1), 3)
                | o
            )
        pltpu.async_copy(table_hbm.at[row_v], rows_v, sem).wait()
        pltpu.sync_copy(rows_v, out_hbm.at[pl.ds(base, b_per_w)])

    return _sc_gather


def _mlp_body(g_ref, nf_ref, w1e_ref, w1n_ref, b1_ref, g1_ref, be1_ref,
              w2_ref, b2_ref, g2_ref, be2_ref, w3_ref, b3_ref, out_ref):
    h = jnp.dot(g_ref[...], w1e_ref[...], preferred_element_type=jnp.float32)
    h = h + jnp.dot(nf_ref[...], w1n_ref[...], preferred_element_type=jnp.float32)
    h = jnp.maximum(h + b1_ref[...], 0.0)
    mean = jnp.mean(h, axis=0, keepdims=True)
    var = jnp.mean((h - mean) ** 2, axis=0, keepdims=True)
    h = g1_ref[...] * ((h - mean) / jnp.sqrt(var + EPS)) + be1_ref[...]
    h = jnp.dot(h, w2_ref[...], preferred_element_type=jnp.float32)
    h = jnp.maximum(h + b2_ref[...], 0.0)
    mean2 = jnp.mean(h, axis=0, keepdims=True)
    var2 = jnp.mean((h - mean2) ** 2, axis=0, keepdims=True)
    h = g2_ref[...] * ((h - mean2) / jnp.sqrt(var2 + EPS)) + be2_ref[...]
    out_ref[...] = (
        jnp.dot(h, w3_ref[...], preferred_element_type=jnp.float32) + b3_ref[...]
    )


def kernel(channel_ids, numeric_features, emb, W1, b1, g1, be1,
           W2, b2, g2, be2, W3, b3):
    emb_t = emb.T
    table_blk = pl.pallas_call(
        _transpose_body,
        grid=(N_TSTEPS,),
        in_specs=[pl.BlockSpec((EMBED_DIM, T_CH), lambda i: (0, i))],
        out_specs=pl.BlockSpec((RUN, BLK), lambda i: (i, 0)),
        out_shape=jax.ShapeDtypeStruct((NUM_BLOCKS, BLK), jnp.float32),
    )(emb_t)
    table_rows = table_blk.reshape(NUM_BLOCKS * ROWS_PER_BLOCK, EMBED_DIM)
    gathered = _make_sc_gather()(table_rows, channel_ids)
    out = pl.pallas_call(
        _mlp_body,
        out_shape=jax.ShapeDtypeStruct((BATCH, 1), jnp.float32),
    )(
        gathered,
        numeric_features,
        W1[:, :EMBED_DIM].T,
        W1[:, EMBED_DIM:].T,
        b1.reshape(1, -1),
        g1.reshape(1, -1),
        be1.reshape(1, -1),
        W2.T,
        b2.reshape(1, -1),
        g2.reshape(1, -1),
        be2.reshape(1, -1),
        W3.T,
        b3.reshape(1, 1),
    )
    return out[:, 0]


# speedup vs baseline: 4.8962x; 1.0087x over previous
import functools

import jax
import jax.numpy as jnp
from jax import lax
from jax.experimental import pallas as pl
from jax.experimental.pallas import tpu as pltpu
from jax.experimental.pallas import tpu_sc as plsc

BATCH = 16384
EMBED_DIM = 16
NUM_NUMERIC = 6
EPS = 1e-5
ROWS_PER_BLOCK = 8
BLK = ROWS_PER_BLOCK * EMBED_DIM
LANES = 16
NUM_CH = 1000000
RUN_SHIFT = 14
RUN = 1 << RUN_SHIFT
T_CH = RUN * ROWS_PER_BLOCK
N_TSTEPS = -(-NUM_CH // T_CH)
NUM_BLOCKS = N_TSTEPS * RUN


def _transpose_body(in_ref, out_ref):
    x = in_ref[...]
    stacked = jnp.concatenate(
        [x[:, o * RUN:(o + 1) * RUN] for o in range(ROWS_PER_BLOCK)], axis=0
    )
    out_ref[...] = stacked.T


@functools.cache
def _make_sc_gather():
    info = plsc.get_sparse_core_info()
    nc, ns = info.num_cores, info.num_subcores
    b_per_w = BATCH // (nc * ns)
    n_groups = b_per_w // LANES
    mesh = plsc.VectorSubcoreMesh(core_axis_name="c", subcore_axis_name="s")

    @functools.partial(
        pl.kernel,
        mesh=mesh,
        out_type=jax.ShapeDtypeStruct((BATCH, EMBED_DIM), jnp.float32),
        scratch_types=[
            pltpu.VMEM((b_per_w,), jnp.int32),
            pltpu.VMEM((b_per_w,), jnp.int32),
            pltpu.VMEM((b_per_w, EMBED_DIM), jnp.float32),
            pltpu.SemaphoreType.DMA,
        ],
        compiler_params=pltpu.CompilerParams(use_tc_tiling_on_sc=False),
    )
    def _sc_gather(table_hbm, idx_hbm, out_hbm, idx_v, row_v, rows_v, sem):
        wid = lax.axis_index("s") * nc + lax.axis_index("c")
        base = wid * b_per_w
        pltpu.sync_copy(idx_hbm.at[pl.ds(base, b_per_w)], idx_v)
        for i in range(n_groups):
            sl = pl.ds(i * LANES, LANES)
            v = idx_v[sl]
            q = lax.shift_right_logical(v, RUN_SHIFT + 3)
            o = lax.shift_right_logical(v, RUN_SHIFT) & (ROWS_PER_BLOCK - 1)
            row_v[sl] = (
                lax.shift_left(q, RUN_SHIFT + 3)
                | lax.shift_left(v & (RUN - 1), 3)
                | o
            )
        pltpu.async_copy(table_hbm.at[row_v], rows_v, sem).wait()
        pltpu.sync_copy(rows_v, out_hbm.at[pl.ds(base, b_per_w)])

    return _sc_gather


def _mlp_body(g_ref, nf_ref, w1e_ref, w1n_ref, b1_ref, g1_ref, be1_ref,
              w2_ref, b2_ref, g2_ref, be2_ref, w3_ref, b3_ref, out_ref):
    h = jnp.dot(g_ref[...], w1e_ref[...], preferred_element_type=jnp.float32)
    h = h + jnp.dot(nf_ref[...], w1n_ref[...], preferred_element_type=jnp.float32)
    h = jnp.maximum(h + b1_ref[...], 0.0)
    mean = jnp.mean(h, axis=0, keepdims=True)
    var = jnp.mean((h - mean) ** 2, axis=0, keepdims=True)
    h = g1_ref[...] * ((h - mean) / jnp.sqrt(var + EPS)) + be1_ref[...]
    h = jnp.dot(h, w2_ref[...], preferred_element_type=jnp.float32)
    h = jnp.maximum(h + b2_ref[...], 0.0)
    mean2 = jnp.mean(h, axis=0, keepdims=True)
    var2 = jnp.mean((h - mean2) ** 2, axis=0, keepdims=True)
    h = g2_ref[...] * ((h - mean2) / jnp.sqrt(var2 + EPS)) + be2_ref[...]
    out_ref[...] = (
        jnp.dot(h, w3_ref[...], preferred_element_type=jnp.float32) + b3_ref[...]
    )


def kernel(channel_ids, numeric_features, emb, W1, b1, g1, be1,
           W2, b2, g2, be2, W3, b3):
    emb_t = emb.T
    table_blk = pl.pallas_call(
        _transpose_body,
        grid=(N_TSTEPS,),
        in_specs=[pl.BlockSpec((EMBED_DIM, T_CH), lambda i: (0, i))],
        out_specs=pl.BlockSpec((RUN, BLK), lambda i: (i, 0)),
        out_shape=jax.ShapeDtypeStruct((NUM_BLOCKS, BLK), jnp.float32),
    )(emb_t)
    table_rows = table_blk.reshape(NUM_BLOCKS * ROWS_PER_BLOCK, EMBED_DIM)
    gathered = _make_sc_gather()(table_rows, channel_ids)
    out = pl.pallas_call(
        _mlp_body,
        out_shape=jax.ShapeDtypeStruct((BATCH, 1), jnp.float32),
    )(
        gathered,
        numeric_features,
        W1[:, :EMBED_DIM].T,
        W1[:, EMBED_DIM:].T,
        b1.reshape(1, -1),
        g1.reshape(1, -1),
        be1.reshape(1, -1),
        W2.T,
        b2.reshape(1, -1),
        g2.reshape(1, -1),
        be2.reshape(1, -1),
        W3.T,
        b3.reshape(1, 1),
    )
    return out[:, 0]
